# baseline (speedup 1.0000x reference)
import functools

import jax
import jax.numpy as jnp
from jax import lax
from jax.experimental import pallas as pl
from jax.experimental.pallas import tpu as pltpu
from jax.experimental.pallas import tpu_sc as plsc

N = 10000
E = 320000
EP = 327680
D = 128
NREL = 26
NP = 10240
NC = 2
NS = 16
NW = NC * NS
EPW = EP // NW
CH = 80
SUP = 1280
RPT = NP // NS
NEG = -1e30

_mesh = plsc.VectorSubcoreMesh(core_axis_name="c", subcore_axis_name="s")


def _t_table(rel_ref, we_ref, ae_ref):
    t26 = rel_ref[...] @ (we_ref[...] @ ae_ref[...])
    return jnp.concatenate(
        [t26, jnp.zeros((128 - NREL, 1), jnp.float32)], axis=0)


def _mlp_prep_body(x_ref, w1_ref, b1_ref, w2_ref, b2_ref, wg_ref, asrc_ref,
                   adst_ref, rel_ref, we_ref, ae_ref,
                   hw_ref, as_ref, ad_ref, t_ref):
    h = jnp.maximum(x_ref[...] @ w1_ref[...] + b1_ref[...], 0.0)
    h = h @ w2_ref[...] + b2_ref[...]
    hw = h @ wg_ref[...]
    hw_ref[...] = hw
    as_ref[...] = (hw @ asrc_ref[...])[:, 0]
    ad_ref[...] = (hw @ adst_ref[...])[:, 0]
    t_ref[...] = _t_table(rel_ref, we_ref, ae_ref)[:, 0]


def _mlp_prep(x, W1, b1, W2, b2, Wg, asrc, adst, rel_emb, We, aedge):
    blk = 2048
    grid = NP // blk
    return pl.pallas_call(
        _mlp_prep_body,
        grid=(grid,),
        in_specs=[
            pl.BlockSpec((blk, D), lambda i: (i, 0)),
            pl.BlockSpec(W1.shape, lambda i: (0, 0)),
            pl.BlockSpec(b1.shape, lambda i: (0,)),
            pl.BlockSpec(W2.shape, lambda i: (0, 0)),
            pl.BlockSpec(b2.shape, lambda i: (0,)),
            pl.BlockSpec(Wg.shape, lambda i: (0, 0)),
            pl.BlockSpec((D, 1), lambda i: (0, 0)),
            pl.BlockSpec((D, 1), lambda i: (0, 0)),
            pl.BlockSpec(rel_emb.shape, lambda i: (0, 0)),
            pl.BlockSpec(We.shape, lambda i: (0, 0)),
            pl.BlockSpec((D, 1), lambda i: (0, 0)),
        ],
        out_specs=[
            pl.BlockSpec((blk, D), lambda i: (i, 0)),
            pl.BlockSpec((blk,), lambda i: (i,)),
            pl.BlockSpec((blk,), lambda i: (i,)),
            pl.BlockSpec((128,), lambda i: (0,)),
        ],
        out_shape=[
            jax.ShapeDtypeStruct((NP, D), jnp.float32),
            jax.ShapeDtypeStruct((NP,), jnp.float32),
            jax.ShapeDtypeStruct((NP,), jnp.float32),
            jax.ShapeDtypeStruct((128,), jnp.float32),
        ],
    )(x, W1, b1, W2, b2, Wg, asrc.reshape(D, 1), adst.reshape(D, 1),
      rel_emb, We, aedge.reshape(D, 1))


def _combine_body(smax_ref, tsum_ref, cnt_ref, as_ref, ad_ref, m_ref, wl_ref):
    cnt = jnp.sum(cnt_ref[...], axis=0)
    tsum = jnp.sum(tsum_ref[...], axis=0)
    smax = jnp.max(smax_ref[...], axis=0)
    sloop = tsum / jnp.maximum(cnt, 1.0)
    al = as_ref[...] + ad_ref[...] + sloop
    al = jnp.where(al >= 0, al, 0.2 * al)
    m = jnp.maximum(smax, al)
    m_ref[...] = m
    wl_ref[...] = jnp.exp(al - m)


def _combine(smax_part, tsum_part, cnt_part, a_src, a_dst):
    return pl.pallas_call(
        _combine_body,
        out_shape=[
            jax.ShapeDtypeStruct((NP,), jnp.float32),
            jax.ShapeDtypeStruct((NP,), jnp.float32),
        ],
    )(smax_part, tsum_part, cnt_part, a_src, a_dst)


def _fin_prep_body(num_ref, den_ref, hw_ref, wl_ref, b_ref, wg_ref, asrc_ref,
                   adst_ref, rel_ref, we_ref, ae_ref,
                   hw2_ref, as_ref, ad_ref, t_ref):
    wl = wl_ref[...][:, None]
    den = jnp.sum(den_ref[...], axis=0)[:, None] + wl
    num = num_ref[0] + num_ref[1] + hw_ref[...] * wl
    h = jnp.maximum(num / den + b_ref[...], 0.0)
    hw2 = h @ wg_ref[...]
    hw2_ref[...] = hw2
    as_ref[...] = (hw2 @ asrc_ref[...])[:, 0]
    ad_ref[...] = (hw2 @ adst_ref[...])[:, 0]
    t_ref[...] = _t_table(rel_ref, we_ref, ae_ref)[:, 0]


def _fin_prep(num_part, den_part, hw, wloop, b, Wg, asrc, adst,
              rel_emb, We, aedge):
    blk = 2048
    grid = NP // blk
    return pl.pallas_call(
        _fin_prep_body,
        grid=(grid,),
        in_specs=[
            pl.BlockSpec((2, blk, D), lambda i: (0, i, 0)),
            pl.BlockSpec((NW, blk), lambda i: (0, i)),
            pl.BlockSpec((blk, D), lambda i: (i, 0)),
            pl.BlockSpec((blk,), lambda i: (i,)),
            pl.BlockSpec(b.shape, lambda i: (0,)),
            pl.BlockSpec(Wg.shape, lambda i: (0, 0)),
            pl.BlockSpec((D, 1), lambda i: (0, 0)),
            pl.BlockSpec((D, 1), lambda i: (0, 0)),
            pl.BlockSpec(rel_emb.shape, lambda i: (0, 0)),
            pl.BlockSpec(We.shape, lambda i: (0, 0)),
            pl.BlockSpec((D, 1), lambda i: (0, 0)),
        ],
        out_specs=[
            pl.BlockSpec((blk, D), lambda i: (i, 0)),
            pl.BlockSpec((blk,), lambda i: (i,)),
            pl.BlockSpec((blk,), lambda i: (i,)),
            pl.BlockSpec((128,), lambda i: (0,)),
        ],
        out_shape=[
            jax.ShapeDtypeStruct((NP, D), jnp.float32),
            jax.ShapeDtypeStruct((NP,), jnp.float32),
            jax.ShapeDtypeStruct((NP,), jnp.float32),
            jax.ShapeDtypeStruct((128,), jnp.float32),
        ],
    )(num_part, den_part, hw, wloop, b, Wg,
      asrc.reshape(D, 1), adst.reshape(D, 1), rel_emb, We, aedge.reshape(D, 1))


def _fin_pool_body(num_ref, den_ref, hw_ref, wl_ref, b_ref, w3_ref, b3_ref,
                   w4_ref, b4_ref, out_ref):
    wl = wl_ref[...][:, None]
    den = jnp.sum(den_ref[...], axis=0)[:, None] + wl
    num = num_ref[0] + num_ref[1] + hw_ref[...] * wl
    h = jnp.maximum(num / den + b_ref[...], 0.0)
    rows = lax.broadcasted_iota(jnp.int32, (NP, 1), 0)
    h = jnp.where(rows < N, h, 0.0)
    pooled = jnp.sum(h, axis=0, keepdims=True) * (1.0 / N)
    out_ref[...] = jnp.maximum(pooled @ w3_ref[...] + b3_ref[...], 0.0) @ w4_ref[...] + b4_ref[...]


def _fin_pool(num_part, den_part, hw, wloop, b, W3, b3, W4, b4):
    return pl.pallas_call(
        _fin_pool_body,
        out_shape=jax.ShapeDtypeStruct((1, 32), jnp.float32),
    )(num_part, den_part, hw, wloop, b, W3, b3, W4, b4)


def _sc_a_body(ei_hbm, attr_hbm, as_hbm, ad_hbm, t_hbm,
               alpha_hbm, smax_hbm, tsum_hbm, cnt_hbm,
               as_v, ad_v, t_v, src_v, dst_v, attr_v, alpha_v,
               smax_v, tsum_v, cnt_v):
    c = lax.axis_index("c")
    s = lax.axis_index("s")
    wid = s * NC + c
    base = wid * EPW

    pltpu.sync_copy(as_hbm, as_v)
    pltpu.sync_copy(ad_hbm, ad_v)
    pltpu.sync_copy(t_hbm, t_v)
    pltpu.sync_copy(ei_hbm.at[0].at[pl.ds(base, EPW)], src_v)
    pltpu.sync_copy(ei_hbm.at[1].at[pl.ds(base, EPW)], dst_v)
    pltpu.sync_copy(attr_hbm.at[pl.ds(base, EPW)], attr_v)

    zero16 = jnp.zeros((16,), jnp.float32)
    zero16i = jnp.zeros((16,), jnp.int32)
    neg16 = jnp.full((16,), NEG, jnp.float32)

    @plsc.parallel_loop(0, NP // 16, 1, unroll=4)
    def init_body(i):
        sl = pl.ds(i * 16, 16)
        smax_v[sl] = neg16
        tsum_v[sl] = zero16
        cnt_v[sl] = zero16

    one16 = jnp.ones((16,), jnp.float32)

    @plsc.parallel_loop(0, EPW // 16, 1, unroll=4)
    def alpha_body(i):
        sl = pl.ds(i * 16, 16)
        s16 = src_v[sl]
        d16 = dst_v[sl]
        a16 = attr_v[sl]
        te = plsc.load_gather(t_v, [a16])
        av = plsc.load_gather(as_v, [s16]) + plsc.load_gather(ad_v, [d16]) + te
        alpha = jnp.where(av >= 0, av, 0.2 * av)
        alpha_v[sl] = alpha
        plsc.addupdate_scatter(cnt_v, [d16], one16)
        plsc.addupdate_scatter(tsum_v, [d16], te)

    def max_body(i, _):
        sl = pl.ds(i * 16, 16)
        d16 = dst_v[sl]
        alpha = alpha_v[sl]
        cur = plsc.load_gather(smax_v, [d16])
        plsc.store_scatter(smax_v, [d16], alpha, mask=alpha > cur)
        cur = plsc.load_gather(smax_v, [d16])

        @pl.when(jnp.any(alpha > cur))
        def _retry():
            def cond(cur_):
                return jnp.any(alpha > cur_)

            def body(cur_):
                plsc.store_scatter(smax_v, [d16], alpha, mask=alpha > cur_)
                return plsc.load_gather(smax_v, [d16])

            lax.while_loop(cond, body, cur)

        return 0

    lax.fori_loop(0, EPW // 16, max_body, 0)

    pltpu.sync_copy(alpha_v, alpha_hbm.at[pl.ds(base, EPW)])
    pltpu.sync_copy(smax_v, smax_hbm.at[wid])
    pltpu.sync_copy(tsum_v, tsum_hbm.at[wid])
    pltpu.sync_copy(cnt_v, cnt_hbm.at[wid])


def _sc_a(edge_index, attr, a_src, a_dst, t):
    f32 = jnp.float32
    return pl.kernel(
        _sc_a_body,
        out_type=[
            jax.ShapeDtypeStruct((EP,), f32),
            jax.ShapeDtypeStruct((NW, NP), f32),
            jax.ShapeDtypeStruct((NW, NP), f32),
            jax.ShapeDtypeStruct((NW, NP), f32),
        ],
        mesh=_mesh,
        compiler_params=pltpu.CompilerParams(needs_layout_passes=False),
        scratch_types=[
            pltpu.VMEM((NP,), f32),
            pltpu.VMEM((NP,), f32),
            pltpu.VMEM((128,), f32),
            pltpu.VMEM((EPW,), jnp.int32),
            pltpu.VMEM((EPW,), jnp.int32),
            pltpu.VMEM((EPW,), jnp.int32),
            pltpu.VMEM((EPW,), f32),
            pltpu.VMEM((NP,), f32),
            pltpu.VMEM((NP,), f32),
            pltpu.VMEM((NP,), f32),
        ],
    )(edge_index, attr, a_src, a_dst, t)


def _sc_b_body(ei_hbm, alpha_hbm, m_hbm, hw_hbm,
               num_hbm, den_hbm,
               m_v, src_v, dst_v, alpha_v, den_v, dstc_a, dstc_b,
               rows_a, rows_b, acc_sh, gsem_a, gsem_b, ssem_a, ssem_b):
    c = lax.axis_index("c")
    s = lax.axis_index("s")
    wid = s * NC + c
    base = wid * EPW

    pltpu.sync_copy(m_hbm, m_v)

    zero16 = jnp.zeros((16,), jnp.float32)
    zero16i = jnp.zeros((16,), jnp.int32)

    def zden_body(i, _):
        den_v[pl.ds(i * 16, 16)] = zero16
        return 0

    lax.fori_loop(0, NP // 16, zden_body, 0)

    def zrow_body(i, _):
        for k in range(D // 16):
            rows_a[i, pl.ds(k * 16, 16)] = zero16
            rows_b[i, pl.ds(k * 16, 16)] = zero16
        return 0

    lax.fori_loop(0, CH, zrow_body, 0)
    for v in range(CH // 16):
        dstc_a[pl.ds(v * 16, 16)] = zero16i
        dstc_b[pl.ds(v * 16, 16)] = zero16i

    def zcopy_body(i, _):
        pltpu.sync_copy(rows_a, acc_sh.at[pl.ds(s * RPT + i * CH, CH)])
        return 0

    lax.fori_loop(0, RPT // CH, zcopy_body, 0)
    plsc.subcore_barrier()

    pltpu.async_copy(rows_a, acc_sh.at[dstc_a], ssem_a, add=True)
    pltpu.async_copy(rows_b, acc_sh.at[dstc_b], ssem_b, add=True)

    splat_idx = [jnp.full((16,), r, jnp.int32) for r in range(16)]

    def stage1(ebase, dstc_v, rows_v, gsem, ssem):
        pltpu.make_async_copy(rows_v, acc_sh.at[dstc_v], ssem).wait()
        gcp = pltpu.async_copy(hw_hbm.at[src_v.at[pl.ds(ebase, CH)]],
                               rows_v, gsem)
        for v in range(CH // 16):
            sl = pl.ds(ebase + v * 16, 16)
            dstc_v[pl.ds(v * 16, 16)] = dst_v[sl]
        return gcp

    def stage2(gcp, ebase, dstc_v, rows_v, ssem):
        gcp.wait()

        @plsc.parallel_loop(0, CH // 16, 1)
        def grp_body(v):
            sl = pl.ds(ebase + v * 16, 16)
            d16 = dst_v[sl]
            w16 = jnp.exp(alpha_v[sl] - plsc.load_gather(m_v, [d16]))
            plsc.addupdate_scatter(den_v, [d16], w16)
            for r in range(16):
                wj = w16.at[splat_idx[r]].get(mode="promise_in_bounds")
                j = v * 16 + r
                for kk in range(D // 16):
                    sl2 = pl.ds(kk * 16, 16)
                    rows_v[j, sl2] = rows_v[j, sl2] * wj

        pltpu.async_copy(rows_v, acc_sh.at[dstc_v], ssem, add=True)

    def super_body(g, _):
        sbase = base + g * SUP
        pltpu.sync_copy(ei_hbm.at[0].at[pl.ds(sbase, SUP)], src_v)
        pltpu.sync_copy(ei_hbm.at[1].at[pl.ds(sbase, SUP)], dst_v)
        pltpu.sync_copy(alpha_hbm.at[pl.ds(sbase, SUP)], alpha_v)

        def pair_body(kp, _):
            e0 = 2 * kp * CH
            gcp_a = stage1(e0, dstc_a, rows_a, gsem_a, ssem_a)
            gcp_b = stage1(e0 + CH, dstc_b, rows_b, gsem_b, ssem_b)
            stage2(gcp_a, e0, dstc_a, rows_a, ssem_a)
            stage2(gcp_b, e0 + CH, dstc_b, rows_b, ssem_b)
            return 0

        lax.fori_loop(0, SUP // CH // 2, pair_body, 0)
        return 0

    lax.fori_loop(0, EPW // SUP, super_body, 0)

    pltpu.make_async_copy(rows_a, acc_sh.at[dstc_a], ssem_a).wait()
    pltpu.make_async_copy(rows_b, acc_sh.at[dstc_b], ssem_b).wait()
    plsc.subcore_barrier()

    pltpu.sync_copy(acc_sh.at[pl.ds(s * RPT, RPT)],
                    num_hbm.at[c].at[pl.ds(s * RPT, RPT)])
    pltpu.sync_copy(den_v, den_hbm.at[wid])


def _sc_b(edge_index, alpha, m, hw):
    f32 = jnp.float32
    return pl.kernel(
        _sc_b_body,
        out_type=[
            jax.ShapeDtypeStruct((NC, NP, D), f32),
            jax.ShapeDtypeStruct((NW, NP), f32),
        ],
        mesh=_mesh,
        compiler_params=pltpu.CompilerParams(needs_layout_passes=False),
        scratch_types=[
            pltpu.VMEM((NP,), f32),
            pltpu.VMEM((SUP,), jnp.int32),
            pltpu.VMEM((SUP,), jnp.int32),
            pltpu.VMEM((SUP,), f32),
            pltpu.VMEM((NP,), f32),
            pltpu.VMEM((CH,), jnp.int32),
            pltpu.VMEM((CH,), jnp.int32),
            pltpu.VMEM((CH, D), f32),
            pltpu.VMEM((CH, D), f32),
            pltpu.VMEM_SHARED((NP, D), f32),
            pltpu.SemaphoreType.DMA,
            pltpu.SemaphoreType.DMA,
            pltpu.SemaphoreType.DMA,
            pltpu.SemaphoreType.DMA,
        ],
    )(edge_index, alpha, m, hw)


def kernel(x, edge_index, edge_attr, W1, b1, W2, b2, rel_emb, c1_W, c1_asrc,
           c1_adst, c1_We, c1_aedge, c1_b, c2_W, c2_asrc, c2_adst, c2_We,
           c2_aedge, c2_b, W3, b3, W4, b4):
    pad_cols = jnp.broadcast_to(
        jnp.array([[0], [N]], jnp.int32), (2, EP - E))
    ei_p = jnp.concatenate([edge_index, pad_cols], axis=1)
    attr_p = jnp.pad(edge_attr, (0, EP - E))
    x_p = jnp.pad(x, ((0, NP - N), (0, 0)))

    hw, a_src, a_dst, t = _mlp_prep(x_p, W1, b1, W2, b2, c1_W, c1_asrc,
                                    c1_adst, rel_emb, c1_We, c1_aedge)

    for li, b in enumerate((c1_b, c2_b)):
        alpha, smax_p, tsum_p, cnt_p = _sc_a(ei_p, attr_p, a_src, a_dst, t)
        m, wloop = _combine(smax_p, tsum_p, cnt_p, a_src, a_dst)
        num_p, den_p = _sc_b(ei_p, alpha, m, hw)
        if li == 0:
            hw, a_src, a_dst, t = _fin_prep(
                num_p, den_p, hw, wloop, b, c2_W, c2_asrc, c2_adst,
                rel_emb, c2_We, c2_aedge)
        else:
            out = _fin_pool(num_p, den_p, hw, wloop, b, W3, b3, W4, b4)
    return out


# speedup vs baseline: 2.4906x; 2.4906x over previous
import functools

import jax
import jax.numpy as jnp
from jax import lax
from jax.experimental import pallas as pl
from jax.experimental.pallas import tpu as pltpu
from jax.experimental.pallas import tpu_sc as plsc

N = 10000
E = 320000
D = 128
NREL = 26
NP = 10240
NC = 2
NS = 16
NW = NC * NS
EPW = E // NW
CH = 80
SUP = 2000
RPT = NP // NS
NEG = -1e30

_mesh = plsc.VectorSubcoreMesh(core_axis_name="c", subcore_axis_name="s")


def _t_table(rel_ref, we_ref, ae_ref):
    t26 = rel_ref[...] @ (we_ref[...] @ ae_ref[...])
    return jnp.concatenate(
        [t26, jnp.zeros((128 - NREL, 1), jnp.float32)], axis=0)


def _mlp_prep_body(x_ref, w1_ref, b1_ref, w2_ref, b2_ref, wg_ref, asrc_ref,
                   adst_ref, rel_ref, we_ref, ae_ref,
                   hw_ref, as_ref, ad_ref, t_ref):
    h = jnp.maximum(x_ref[...] @ w1_ref[...] + b1_ref[...], 0.0)
    h = h @ w2_ref[...] + b2_ref[...]
    hw = h @ wg_ref[...]
    hw_ref[...] = hw
    as_ref[...] = (hw @ asrc_ref[...])[:, 0]
    ad_ref[...] = (hw @ adst_ref[...])[:, 0]
    t_ref[...] = _t_table(rel_ref, we_ref, ae_ref)[:, 0]


def _mlp_prep(x, W1, b1, W2, b2, Wg, asrc, adst, rel_emb, We, aedge):
    blk = 2048
    grid = NP // blk
    return pl.pallas_call(
        _mlp_prep_body,
        grid=(grid,),
        in_specs=[
            pl.BlockSpec((blk, D), lambda i: (i, 0)),
            pl.BlockSpec(W1.shape, lambda i: (0, 0)),
            pl.BlockSpec(b1.shape, lambda i: (0,)),
            pl.BlockSpec(W2.shape, lambda i: (0, 0)),
            pl.BlockSpec(b2.shape, lambda i: (0,)),
            pl.BlockSpec(Wg.shape, lambda i: (0, 0)),
            pl.BlockSpec((D, 1), lambda i: (0, 0)),
            pl.BlockSpec((D, 1), lambda i: (0, 0)),
            pl.BlockSpec(rel_emb.shape, lambda i: (0, 0)),
            pl.BlockSpec(We.shape, lambda i: (0, 0)),
            pl.BlockSpec((D, 1), lambda i: (0, 0)),
        ],
        out_specs=[
            pl.BlockSpec((blk, D), lambda i: (i, 0)),
            pl.BlockSpec((blk,), lambda i: (i,)),
            pl.BlockSpec((blk,), lambda i: (i,)),
            pl.BlockSpec((128,), lambda i: (0,)),
        ],
        out_shape=[
            jax.ShapeDtypeStruct((NP, D), jnp.float32),
            jax.ShapeDtypeStruct((NP,), jnp.float32),
            jax.ShapeDtypeStruct((NP,), jnp.float32),
            jax.ShapeDtypeStruct((128,), jnp.float32),
        ],
    )(x, W1, b1, W2, b2, Wg, asrc.reshape(D, 1), adst.reshape(D, 1),
      rel_emb, We, aedge.reshape(D, 1))


def _combine_body(smax_ref, tsum_ref, cnt_ref, as_ref, ad_ref, m_ref, wl_ref):
    cnt = jnp.sum(cnt_ref[...], axis=0)
    tsum = jnp.sum(tsum_ref[...], axis=0)
    smax = jnp.max(smax_ref[...], axis=0)
    sloop = tsum / jnp.maximum(cnt, 1.0)
    al = as_ref[...] + ad_ref[...] + sloop
    al = jnp.where(al >= 0, al, 0.2 * al)
    m = jnp.maximum(smax, al)
    m_ref[...] = m
    wl_ref[...] = jnp.exp(al - m)


def _combine(smax_part, tsum_part, cnt_part, a_src, a_dst):
    return pl.pallas_call(
        _combine_body,
        out_shape=[
            jax.ShapeDtypeStruct((NP,), jnp.float32),
            jax.ShapeDtypeStruct((NP,), jnp.float32),
        ],
    )(smax_part, tsum_part, cnt_part, a_src, a_dst)


def _fin_prep_body(num_ref, den_ref, hw_ref, wl_ref, b_ref, wg_ref, asrc_ref,
                   adst_ref, rel_ref, we_ref, ae_ref,
                   hw2_ref, as_ref, ad_ref, t_ref):
    wl = wl_ref[...][:, None]
    den = jnp.sum(den_ref[...], axis=0)[:, None] + wl
    num = num_ref[0] + num_ref[1] + hw_ref[...] * wl
    h = jnp.maximum(num / den + b_ref[...], 0.0)
    hw2 = h @ wg_ref[...]
    hw2_ref[...] = hw2
    as_ref[...] = (hw2 @ asrc_ref[...])[:, 0]
    ad_ref[...] = (hw2 @ adst_ref[...])[:, 0]
    t_ref[...] = _t_table(rel_ref, we_ref, ae_ref)[:, 0]


def _fin_prep(num_part, den_part, hw, wloop, b, Wg, asrc, adst,
              rel_emb, We, aedge):
    blk = 2048
    grid = NP // blk
    return pl.pallas_call(
        _fin_prep_body,
        grid=(grid,),
        in_specs=[
            pl.BlockSpec((2, blk, D), lambda i: (0, i, 0)),
            pl.BlockSpec((NW, blk), lambda i: (0, i)),
            pl.BlockSpec((blk, D), lambda i: (i, 0)),
            pl.BlockSpec((blk,), lambda i: (i,)),
            pl.BlockSpec(b.shape, lambda i: (0,)),
            pl.BlockSpec(Wg.shape, lambda i: (0, 0)),
            pl.BlockSpec((D, 1), lambda i: (0, 0)),
            pl.BlockSpec((D, 1), lambda i: (0, 0)),
            pl.BlockSpec(rel_emb.shape, lambda i: (0, 0)),
            pl.BlockSpec(We.shape, lambda i: (0, 0)),
            pl.BlockSpec((D, 1), lambda i: (0, 0)),
        ],
        out_specs=[
            pl.BlockSpec((blk, D), lambda i: (i, 0)),
            pl.BlockSpec((blk,), lambda i: (i,)),
            pl.BlockSpec((blk,), lambda i: (i,)),
            pl.BlockSpec((128,), lambda i: (0,)),
        ],
        out_shape=[
            jax.ShapeDtypeStruct((NP, D), jnp.float32),
            jax.ShapeDtypeStruct((NP,), jnp.float32),
            jax.ShapeDtypeStruct((NP,), jnp.float32),
            jax.ShapeDtypeStruct((128,), jnp.float32),
        ],
    )(num_part, den_part, hw, wloop, b, Wg,
      asrc.reshape(D, 1), adst.reshape(D, 1), rel_emb, We, aedge.reshape(D, 1))


def _fin_pool_body(num_ref, den_ref, hw_ref, wl_ref, b_ref, w3_ref, b3_ref,
                   w4_ref, b4_ref, out_ref):
    wl = wl_ref[...][:, None]
    den = jnp.sum(den_ref[...], axis=0)[:, None] + wl
    num = num_ref[0] + num_ref[1] + hw_ref[...] * wl
    h = jnp.maximum(num / den + b_ref[...], 0.0)
    rows = lax.broadcasted_iota(jnp.int32, (NP, 1), 0)
    h = jnp.where(rows < N, h, 0.0)
    pooled = jnp.sum(h, axis=0, keepdims=True) * (1.0 / N)
    out_ref[...] = jnp.maximum(pooled @ w3_ref[...] + b3_ref[...], 0.0) @ w4_ref[...] + b4_ref[...]


def _fin_pool(num_part, den_part, hw, wloop, b, W3, b3, W4, b4):
    return pl.pallas_call(
        _fin_pool_body,
        out_shape=jax.ShapeDtypeStruct((1, 32), jnp.float32),
    )(num_part, den_part, hw, wloop, b, W3, b3, W4, b4)


def _sc_a_body(src_hbm, dst_hbm, attr_hbm, as_hbm, ad_hbm, t_hbm,
               alpha_hbm, smax_hbm, tsum_hbm, cnt_hbm,
               as_v, ad_v, t_v, src_v, dst_v, attr_v, alpha_v,
               smax_v, tsum_v, cnt_v):
    c = lax.axis_index("c")
    s = lax.axis_index("s")
    wid = s * NC + c
    base = wid * EPW

    pltpu.sync_copy(as_hbm, as_v)
    pltpu.sync_copy(ad_hbm, ad_v)
    pltpu.sync_copy(t_hbm, t_v)
    pltpu.sync_copy(src_hbm.at[pl.ds(base, EPW)], src_v)
    pltpu.sync_copy(dst_hbm.at[pl.ds(base, EPW)], dst_v)
    pltpu.sync_copy(attr_hbm.at[pl.ds(base, EPW)], attr_v)

    zero16 = jnp.zeros((16,), jnp.float32)
    zero16i = jnp.zeros((16,), jnp.int32)
    neg16 = jnp.full((16,), NEG, jnp.float32)

    @plsc.parallel_loop(0, NP // 16, 1, unroll=4)
    def init_body(i):
        sl = pl.ds(i * 16, 16)
        smax_v[sl] = neg16
        tsum_v[sl] = zero16
        cnt_v[sl] = zero16

    one16 = jnp.ones((16,), jnp.float32)

    @plsc.parallel_loop(0, EPW // 16, 1, unroll=4)
    def alpha_body(i):
        sl = pl.ds(i * 16, 16)
        s16 = src_v[sl]
        d16 = dst_v[sl]
        a16 = attr_v[sl]
        te = plsc.load_gather(t_v, [a16])
        av = plsc.load_gather(as_v, [s16]) + plsc.load_gather(ad_v, [d16]) + te
        alpha = jnp.where(av >= 0, av, 0.2 * av)
        alpha_v[sl] = alpha
        plsc.addupdate_scatter(cnt_v, [d16], one16)
        plsc.addupdate_scatter(tsum_v, [d16], te)

    def max_body(i, _):
        sl = pl.ds(i * 16, 16)
        d16 = dst_v[sl]
        alpha = alpha_v[sl]
        cur = plsc.load_gather(smax_v, [d16])
        plsc.store_scatter(smax_v, [d16], alpha, mask=alpha > cur)
        cur = plsc.load_gather(smax_v, [d16])

        @pl.when(jnp.any(alpha > cur))
        def _retry():
            def cond(cur_):
                return jnp.any(alpha > cur_)

            def body(cur_):
                plsc.store_scatter(smax_v, [d16], alpha, mask=alpha > cur_)
                return plsc.load_gather(smax_v, [d16])

            lax.while_loop(cond, body, cur)

        return 0

    lax.fori_loop(0, EPW // 16, max_body, 0)

    pltpu.sync_copy(alpha_v, alpha_hbm.at[pl.ds(base, EPW)])
    pltpu.sync_copy(smax_v, smax_hbm.at[wid])
    pltpu.sync_copy(tsum_v, tsum_hbm.at[wid])
    pltpu.sync_copy(cnt_v, cnt_hbm.at[wid])


def _sc_a(src, dst, attr, a_src, a_dst, t):
    f32 = jnp.float32
    return pl.kernel(
        _sc_a_body,
        out_type=[
            jax.ShapeDtypeStruct((E,), f32),
            jax.ShapeDtypeStruct((NW, NP), f32),
            jax.ShapeDtypeStruct((NW, NP), f32),
            jax.ShapeDtypeStruct((NW, NP), f32),
        ],
        mesh=_mesh,
        compiler_params=pltpu.CompilerParams(needs_layout_passes=False),
        scratch_types=[
            pltpu.VMEM((NP,), f32),
            pltpu.VMEM((NP,), f32),
            pltpu.VMEM((128,), f32),
            pltpu.VMEM((EPW,), jnp.int32),
            pltpu.VMEM((EPW,), jnp.int32),
            pltpu.VMEM((EPW,), jnp.int32),
            pltpu.VMEM((EPW,), f32),
            pltpu.VMEM((NP,), f32),
            pltpu.VMEM((NP,), f32),
            pltpu.VMEM((NP,), f32),
        ],
    )(src, dst, attr, a_src, a_dst, t)


def _sc_b_body(src_hbm, dst_hbm, alpha_hbm, m_hbm, hw_hbm,
               num_hbm, den_hbm,
               m_v, src_v, dst_v, alpha_v, den_v, w_a, w_b, dstc_a, dstc_b,
               rows_a, rows_b, acc_sh, gsem_a, gsem_b, ssem_a, ssem_b):
    c = lax.axis_index("c")
    s = lax.axis_index("s")
    wid = s * NC + c
    base = wid * EPW

    pltpu.sync_copy(m_hbm, m_v)

    zero16 = jnp.zeros((16,), jnp.float32)
    zero16i = jnp.zeros((16,), jnp.int32)

    def zden_body(i, _):
        den_v[pl.ds(i * 16, 16)] = zero16
        return 0

    lax.fori_loop(0, NP // 16, zden_body, 0)

    def zrow_body(i, _):
        for k in range(D // 16):
            rows_a[i, pl.ds(k * 16, 16)] = zero16
            rows_b[i, pl.ds(k * 16, 16)] = zero16
        return 0

    lax.fori_loop(0, CH, zrow_body, 0)
    for v in range(CH // 16):
        dstc_a[pl.ds(v * 16, 16)] = zero16i
        dstc_b[pl.ds(v * 16, 16)] = zero16i

    def zcopy_body(i, _):
        pltpu.sync_copy(rows_a, acc_sh.at[pl.ds(s * RPT + i * CH, CH)])
        return 0

    lax.fori_loop(0, RPT // CH, zcopy_body, 0)
    plsc.subcore_barrier()

    pltpu.async_copy(rows_a, acc_sh.at[dstc_a], ssem_a, add=True)
    pltpu.async_copy(rows_b, acc_sh.at[dstc_b], ssem_b, add=True)

    def stage1(ebase, w_v, dstc_v, rows_v, gsem, ssem):
        pltpu.make_async_copy(rows_v, acc_sh.at[dstc_v], ssem).wait()
        gcp = pltpu.async_copy(hw_hbm.at[src_v.at[pl.ds(ebase, CH)]],
                               rows_v, gsem)
        for v in range(CH // 16):
            sl = pl.ds(ebase + v * 16, 16)
            d16 = dst_v[sl]
            w = jnp.exp(alpha_v[sl] - plsc.load_gather(m_v, [d16]))
            w_v[pl.ds(v * 16, 16)] = w
            dstc_v[pl.ds(v * 16, 16)] = d16
            plsc.addupdate_scatter(den_v, [d16], w)
        return gcp

    def stage2(gcp, w_v, dstc_v, rows_v, ssem):
        gcp.wait()

        @plsc.parallel_loop(0, CH, 1, unroll=4)
        def mul_body(j):
            wj = plsc.load_gather(w_v, [jnp.full((16,), j, jnp.int32)])
            for kk in range(D // 16):
                sl2 = pl.ds(kk * 16, 16)
                rows_v[j, sl2] = rows_v[j, sl2] * wj

        pltpu.async_copy(rows_v, acc_sh.at[dstc_v], ssem, add=True)

    def super_body(g, _):
        sbase = base + g * SUP
        pltpu.sync_copy(src_hbm.at[pl.ds(sbase, SUP)], src_v)
        pltpu.sync_copy(dst_hbm.at[pl.ds(sbase, SUP)], dst_v)
        pltpu.sync_copy(alpha_hbm.at[pl.ds(sbase, SUP)], alpha_v)

        gcp = stage1(0, w_a, dstc_a, rows_a, gsem_a, ssem_a)
        stage2(gcp, w_a, dstc_a, rows_a, ssem_a)

        def pair_body(kp, _):
            e0 = (1 + 2 * kp) * CH
            gcp_a = stage1(e0, w_a, dstc_a, rows_a, gsem_a, ssem_a)
            gcp_b = stage1(e0 + CH, w_b, dstc_b, rows_b, gsem_b, ssem_b)
            stage2(gcp_a, w_a, dstc_a, rows_a, ssem_a)
            stage2(gcp_b, w_b, dstc_b, rows_b, ssem_b)
            return 0

        lax.fori_loop(0, (SUP // CH - 1) // 2, pair_body, 0)
        return 0

    lax.fori_loop(0, EPW // SUP, super_body, 0)

    pltpu.make_async_copy(rows_a, acc_sh.at[dstc_a], ssem_a).wait()
    pltpu.make_async_copy(rows_b, acc_sh.at[dstc_b], ssem_b).wait()
    plsc.subcore_barrier()

    pltpu.sync_copy(acc_sh.at[pl.ds(s * RPT, RPT)],
                    num_hbm.at[c].at[pl.ds(s * RPT, RPT)])
    pltpu.sync_copy(den_v, den_hbm.at[wid])


def _sc_b(src, dst, alpha, m, hw):
    f32 = jnp.float32
    return pl.kernel(
        _sc_b_body,
        out_type=[
            jax.ShapeDtypeStruct((NC, NP, D), f32),
            jax.ShapeDtypeStruct((NW, NP), f32),
        ],
        mesh=_mesh,
        compiler_params=pltpu.CompilerParams(needs_layout_passes=False),
        scratch_types=[
            pltpu.VMEM((NP,), f32),
            pltpu.VMEM((SUP,), jnp.int32),
            pltpu.VMEM((SUP,), jnp.int32),
            pltpu.VMEM((SUP,), f32),
            pltpu.VMEM((NP,), f32),
            pltpu.VMEM((CH,), f32),
            pltpu.VMEM((CH,), f32),
            pltpu.VMEM((CH,), jnp.int32),
            pltpu.VMEM((CH,), jnp.int32),
            pltpu.VMEM((CH, D), f32),
            pltpu.VMEM((CH, D), f32),
            pltpu.VMEM_SHARED((NP, D), f32),
            pltpu.SemaphoreType.DMA,
            pltpu.SemaphoreType.DMA,
            pltpu.SemaphoreType.DMA,
            pltpu.SemaphoreType.DMA,
        ],
    )(src, dst, alpha, m, hw)


def kernel(x, edge_index, edge_attr, W1, b1, W2, b2, rel_emb, c1_W, c1_asrc,
           c1_adst, c1_We, c1_aedge, c1_b, c2_W, c2_asrc, c2_adst, c2_We,
           c2_aedge, c2_b, W3, b3, W4, b4):
    src = edge_index[0]
    dst = edge_index[1]
    x_p = jnp.pad(x, ((0, NP - N), (0, 0)))

    hw, a_src, a_dst, t = _mlp_prep(x_p, W1, b1, W2, b2, c1_W, c1_asrc,
                                    c1_adst, rel_emb, c1_We, c1_aedge)

    for li, b in enumerate((c1_b, c2_b)):
        alpha, smax_p, tsum_p, cnt_p = _sc_a(
            src, dst, edge_attr, a_src, a_dst, t)
        m, wloop = _combine(smax_p, tsum_p, cnt_p, a_src, a_dst)
        num_p, den_p = _sc_b(src, dst, alpha, m, hw)
        if li == 0:
            hw, a_src, a_dst, t = _fin_prep(
                num_p, den_p, hw, wloop, b, c2_W, c2_asrc, c2_adst,
                rel_emb, c2_We, c2_aedge)
        else:
            out = _fin_pool(num_p, den_p, hw, wloop, b, W3, b3, W4, b4)
    return out


# speedup vs baseline: 2.5502x; 1.0239x over previous
import functools

import jax
import jax.numpy as jnp
from jax import lax
from jax.experimental import pallas as pl
from jax.experimental.pallas import tpu as pltpu
from jax.experimental.pallas import tpu_sc as plsc

N = 10000
E = 320000
D = 128
NREL = 26
NP = 10240
NC = 2
NS = 16
NW = NC * NS
EPW = E // NW
CH = 80
SUP = 2000
RPT = NP // NS
NEG = -1e30

_mesh = plsc.VectorSubcoreMesh(core_axis_name="c", subcore_axis_name="s")


def _t_table(rel_ref, we_ref, ae_ref):
    t26 = rel_ref[...] @ (we_ref[...] @ ae_ref[...])
    return jnp.concatenate(
        [t26, jnp.zeros((128 - NREL, 1), jnp.float32)], axis=0)


def _mlp_prep_body(x_ref, w1_ref, b1_ref, w2_ref, b2_ref, wg_ref, asrc_ref,
                   adst_ref, rel_ref, we_ref, ae_ref,
                   hw_ref, as_ref, ad_ref, t_ref):
    h = jnp.maximum(x_ref[...] @ w1_ref[...] + b1_ref[...], 0.0)
    h = h @ w2_ref[...] + b2_ref[...]
    hw = h @ wg_ref[...]
    hw_ref[...] = hw
    as_ref[...] = (hw @ asrc_ref[...])[:, 0]
    ad_ref[...] = (hw @ adst_ref[...])[:, 0]
    t_ref[...] = _t_table(rel_ref, we_ref, ae_ref)[:, 0]


def _mlp_prep(x, W1, b1, W2, b2, Wg, asrc, adst, rel_emb, We, aedge):
    blk = 2048
    grid = NP // blk
    return pl.pallas_call(
        _mlp_prep_body,
        grid=(grid,),
        in_specs=[
            pl.BlockSpec((blk, D), lambda i: (i, 0)),
            pl.BlockSpec(W1.shape, lambda i: (0, 0)),
            pl.BlockSpec(b1.shape, lambda i: (0,)),
            pl.BlockSpec(W2.shape, lambda i: (0, 0)),
            pl.BlockSpec(b2.shape, lambda i: (0,)),
            pl.BlockSpec(Wg.shape, lambda i: (0, 0)),
            pl.BlockSpec((D, 1), lambda i: (0, 0)),
            pl.BlockSpec((D, 1), lambda i: (0, 0)),
            pl.BlockSpec(rel_emb.shape, lambda i: (0, 0)),
            pl.BlockSpec(We.shape, lambda i: (0, 0)),
            pl.BlockSpec((D, 1), lambda i: (0, 0)),
        ],
        out_specs=[
            pl.BlockSpec((blk, D), lambda i: (i, 0)),
            pl.BlockSpec((blk,), lambda i: (i,)),
            pl.BlockSpec((blk,), lambda i: (i,)),
            pl.BlockSpec((128,), lambda i: (0,)),
        ],
        out_shape=[
            jax.ShapeDtypeStruct((NP, D), jnp.float32),
            jax.ShapeDtypeStruct((NP,), jnp.float32),
            jax.ShapeDtypeStruct((NP,), jnp.float32),
            jax.ShapeDtypeStruct((128,), jnp.float32),
        ],
    )(x, W1, b1, W2, b2, Wg, asrc.reshape(D, 1), adst.reshape(D, 1),
      rel_emb, We, aedge.reshape(D, 1))


def _combine_body(smax_ref, tsum_ref, cnt_ref, as_ref, ad_ref, m_ref, wl_ref):
    cnt = jnp.sum(cnt_ref[...], axis=0)
    tsum = jnp.sum(tsum_ref[...], axis=0)
    smax = jnp.max(smax_ref[...], axis=0)
    sloop = tsum / jnp.maximum(cnt, 1.0)
    al = as_ref[...] + ad_ref[...] + sloop
    al = jnp.where(al >= 0, al, 0.2 * al)
    m = jnp.maximum(smax, al)
    m_ref[...] = m
    wl_ref[...] = jnp.exp(al - m)


def _combine(smax_part, tsum_part, cnt_part, a_src, a_dst):
    return pl.pallas_call(
        _combine_body,
        out_shape=[
            jax.ShapeDtypeStruct((NP,), jnp.float32),
            jax.ShapeDtypeStruct((NP,), jnp.float32),
        ],
    )(smax_part, tsum_part, cnt_part, a_src, a_dst)


def _fin_prep_body(num_ref, den_ref, hw_ref, wl_ref, b_ref, wg_ref, asrc_ref,
                   adst_ref, rel_ref, we_ref, ae_ref,
                   hw2_ref, as_ref, ad_ref, t_ref):
    wl = wl_ref[...][:, None]
    den = jnp.sum(den_ref[...], axis=0)[:, None] + wl
    num = num_ref[0] + num_ref[1] + hw_ref[...] * wl
    h = jnp.maximum(num / den + b_ref[...], 0.0)
    hw2 = h @ wg_ref[...]
    hw2_ref[...] = hw2
    as_ref[...] = (hw2 @ asrc_ref[...])[:, 0]
    ad_ref[...] = (hw2 @ adst_ref[...])[:, 0]
    t_ref[...] = _t_table(rel_ref, we_ref, ae_ref)[:, 0]


def _fin_prep(num_part, den_part, hw, wloop, b, Wg, asrc, adst,
              rel_emb, We, aedge):
    blk = 2048
    grid = NP // blk
    return pl.pallas_call(
        _fin_prep_body,
        grid=(grid,),
        in_specs=[
            pl.BlockSpec((2, blk, D), lambda i: (0, i, 0)),
            pl.BlockSpec((NW, blk), lambda i: (0, i)),
            pl.BlockSpec((blk, D), lambda i: (i, 0)),
            pl.BlockSpec((blk,), lambda i: (i,)),
            pl.BlockSpec(b.shape, lambda i: (0,)),
            pl.BlockSpec(Wg.shape, lambda i: (0, 0)),
            pl.BlockSpec((D, 1), lambda i: (0, 0)),
            pl.BlockSpec((D, 1), lambda i: (0, 0)),
            pl.BlockSpec(rel_emb.shape, lambda i: (0, 0)),
            pl.BlockSpec(We.shape, lambda i: (0, 0)),
            pl.BlockSpec((D, 1), lambda i: (0, 0)),
        ],
        out_specs=[
            pl.BlockSpec((blk, D), lambda i: (i, 0)),
            pl.BlockSpec((blk,), lambda i: (i,)),
            pl.BlockSpec((blk,), lambda i: (i,)),
            pl.BlockSpec((128,), lambda i: (0,)),
        ],
        out_shape=[
            jax.ShapeDtypeStruct((NP, D), jnp.float32),
            jax.ShapeDtypeStruct((NP,), jnp.float32),
            jax.ShapeDtypeStruct((NP,), jnp.float32),
            jax.ShapeDtypeStruct((128,), jnp.float32),
        ],
    )(num_part, den_part, hw, wloop, b, Wg,
      asrc.reshape(D, 1), adst.reshape(D, 1), rel_emb, We, aedge.reshape(D, 1))


def _fin_pool_body(num_ref, den_ref, hw_ref, wl_ref, b_ref, w3_ref, b3_ref,
                   w4_ref, b4_ref, out_ref):
    wl = wl_ref[...][:, None]
    den = jnp.sum(den_ref[...], axis=0)[:, None] + wl
    num = num_ref[0] + num_ref[1] + hw_ref[...] * wl
    h = jnp.maximum(num / den + b_ref[...], 0.0)
    rows = lax.broadcasted_iota(jnp.int32, (NP, 1), 0)
    h = jnp.where(rows < N, h, 0.0)
    pooled = jnp.sum(h, axis=0, keepdims=True) * (1.0 / N)
    out_ref[...] = jnp.maximum(pooled @ w3_ref[...] + b3_ref[...], 0.0) @ w4_ref[...] + b4_ref[...]


def _fin_pool(num_part, den_part, hw, wloop, b, W3, b3, W4, b4):
    return pl.pallas_call(
        _fin_pool_body,
        out_shape=jax.ShapeDtypeStruct((1, 32), jnp.float32),
    )(num_part, den_part, hw, wloop, b, W3, b3, W4, b4)


def _sc_a_body(ei_hbm, attr_hbm, as_hbm, ad_hbm, t_hbm,
               alpha_hbm, smax_hbm, tsum_hbm, cnt_hbm,
               as_v, ad_v, t_v, src_v, dst_v, attr_v, alpha_v,
               smax_v, tsum_v, cnt_v):
    c = lax.axis_index("c")
    s = lax.axis_index("s")
    wid = s * NC + c
    base = wid * EPW

    pltpu.sync_copy(as_hbm, as_v)
    pltpu.sync_copy(ad_hbm, ad_v)
    pltpu.sync_copy(t_hbm, t_v)
    pltpu.sync_copy(ei_hbm.at[pl.ds(base, EPW)], src_v)
    pltpu.sync_copy(ei_hbm.at[pl.ds(E + base, EPW)], dst_v)
    pltpu.sync_copy(attr_hbm.at[pl.ds(base, EPW)], attr_v)

    zero16 = jnp.zeros((16,), jnp.float32)
    zero16i = jnp.zeros((16,), jnp.int32)
    neg16 = jnp.full((16,), NEG, jnp.float32)

    @plsc.parallel_loop(0, NP // 16, 1, unroll=4)
    def init_body(i):
        sl = pl.ds(i * 16, 16)
        smax_v[sl] = neg16
        tsum_v[sl] = zero16
        cnt_v[sl] = zero16

    one16 = jnp.ones((16,), jnp.float32)

    @plsc.parallel_loop(0, EPW // 16, 1, unroll=4)
    def alpha_body(i):
        sl = pl.ds(i * 16, 16)
        s16 = src_v[sl]
        d16 = dst_v[sl]
        a16 = attr_v[sl]
        te = plsc.load_gather(t_v, [a16])
        av = plsc.load_gather(as_v, [s16]) + plsc.load_gather(ad_v, [d16]) + te
        alpha = jnp.where(av >= 0, av, 0.2 * av)
        alpha_v[sl] = alpha
        plsc.addupdate_scatter(cnt_v, [d16], one16)
        plsc.addupdate_scatter(tsum_v, [d16], te)

    def max_body(i, _):
        sl = pl.ds(i * 16, 16)
        d16 = dst_v[sl]
        alpha = alpha_v[sl]
        cur = plsc.load_gather(smax_v, [d16])
        plsc.store_scatter(smax_v, [d16], alpha, mask=alpha > cur)
        cur = plsc.load_gather(smax_v, [d16])

        @pl.when(jnp.any(alpha > cur))
        def _retry():
            def cond(cur_):
                return jnp.any(alpha > cur_)

            def body(cur_):
                plsc.store_scatter(smax_v, [d16], alpha, mask=alpha > cur_)
                return plsc.load_gather(smax_v, [d16])

            lax.while_loop(cond, body, cur)

        return 0

    lax.fori_loop(0, EPW // 16, max_body, 0)

    pltpu.sync_copy(alpha_v, alpha_hbm.at[pl.ds(base, EPW)])
    pltpu.sync_copy(smax_v, smax_hbm.at[wid])
    pltpu.sync_copy(tsum_v, tsum_hbm.at[wid])
    pltpu.sync_copy(cnt_v, cnt_hbm.at[wid])


def _sc_a(ei_flat, attr, a_src, a_dst, t):
    f32 = jnp.float32
    return pl.kernel(
        _sc_a_body,
        out_type=[
            jax.ShapeDtypeStruct((E,), f32),
            jax.ShapeDtypeStruct((NW, NP), f32),
            jax.ShapeDtypeStruct((NW, NP), f32),
            jax.ShapeDtypeStruct((NW, NP), f32),
        ],
        mesh=_mesh,
        compiler_params=pltpu.CompilerParams(needs_layout_passes=False),
        scratch_types=[
            pltpu.VMEM((NP,), f32),
            pltpu.VMEM((NP,), f32),
            pltpu.VMEM((128,), f32),
            pltpu.VMEM((EPW,), jnp.int32),
            pltpu.VMEM((EPW,), jnp.int32),
            pltpu.VMEM((EPW,), jnp.int32),
            pltpu.VMEM((EPW,), f32),
            pltpu.VMEM((NP,), f32),
            pltpu.VMEM((NP,), f32),
            pltpu.VMEM((NP,), f32),
        ],
    )(ei_flat, attr, a_src, a_dst, t)


def _sc_b_body(ei_hbm, alpha_hbm, m_hbm, hw_hbm,
               num_hbm, den_hbm,
               m_v, src_v, dst_v, alpha_v, den_v, w_a, w_b, dstc_a, dstc_b,
               rows_a, rows_b, acc_sh, gsem_a, gsem_b, ssem_a, ssem_b):
    c = lax.axis_index("c")
    s = lax.axis_index("s")
    wid = s * NC + c
    base = wid * EPW

    pltpu.sync_copy(m_hbm, m_v)

    zero16 = jnp.zeros((16,), jnp.float32)
    zero16i = jnp.zeros((16,), jnp.int32)

    def zden_body(i, _):
        den_v[pl.ds(i * 16, 16)] = zero16
        return 0

    lax.fori_loop(0, NP // 16, zden_body, 0)

    def zrow_body(i, _):
        for k in range(D // 16):
            rows_a[i, pl.ds(k * 16, 16)] = zero16
            rows_b[i, pl.ds(k * 16, 16)] = zero16
        return 0

    lax.fori_loop(0, CH, zrow_body, 0)
    for v in range(CH // 16):
        dstc_a[pl.ds(v * 16, 16)] = zero16i
        dstc_b[pl.ds(v * 16, 16)] = zero16i

    def zcopy_body(i, _):
        pltpu.sync_copy(rows_a, acc_sh.at[pl.ds(s * RPT + i * CH, CH)])
        return 0

    lax.fori_loop(0, RPT // CH, zcopy_body, 0)
    plsc.subcore_barrier()

    pltpu.async_copy(rows_a, acc_sh.at[dstc_a], ssem_a, add=True)
    pltpu.async_copy(rows_b, acc_sh.at[dstc_b], ssem_b, add=True)

    def stage1(ebase, w_v, dstc_v, rows_v, gsem, ssem):
        pltpu.make_async_copy(rows_v, acc_sh.at[dstc_v], ssem).wait()
        gcp = pltpu.async_copy(hw_hbm.at[src_v.at[pl.ds(ebase, CH)]],
                               rows_v, gsem)
        for v in range(CH // 16):
            sl = pl.ds(ebase + v * 16, 16)
            d16 = dst_v[sl]
            w = jnp.exp(alpha_v[sl] - plsc.load_gather(m_v, [d16]))
            w_v[pl.ds(v * 16, 16)] = w
            dstc_v[pl.ds(v * 16, 16)] = d16
            plsc.addupdate_scatter(den_v, [d16], w)
        return gcp

    def stage2(gcp, w_v, dstc_v, rows_v, ssem):
        gcp.wait()

        @plsc.parallel_loop(0, CH, 1, unroll=4)
        def mul_body(j):
            wj = plsc.load_gather(w_v, [jnp.full((16,), j, jnp.int32)])
            for kk in range(D // 16):
                sl2 = pl.ds(kk * 16, 16)
                rows_v[j, sl2] = rows_v[j, sl2] * wj

        pltpu.async_copy(rows_v, acc_sh.at[dstc_v], ssem, add=True)

    def super_body(g, _):
        sbase = base + g * SUP
        pltpu.sync_copy(ei_hbm.at[pl.ds(sbase, SUP)], src_v)
        pltpu.sync_copy(ei_hbm.at[pl.ds(E + sbase, SUP)], dst_v)
        pltpu.sync_copy(alpha_hbm.at[pl.ds(sbase, SUP)], alpha_v)

        gcp = stage1(0, w_a, dstc_a, rows_a, gsem_a, ssem_a)
        stage2(gcp, w_a, dstc_a, rows_a, ssem_a)

        def pair_body(kp, _):
            e0 = (1 + 2 * kp) * CH
            gcp_a = stage1(e0, w_a, dstc_a, rows_a, gsem_a, ssem_a)
            gcp_b = stage1(e0 + CH, w_b, dstc_b, rows_b, gsem_b, ssem_b)
            stage2(gcp_a, w_a, dstc_a, rows_a, ssem_a)
            stage2(gcp_b, w_b, dstc_b, rows_b, ssem_b)
            return 0

        lax.fori_loop(0, (SUP // CH - 1) // 2, pair_body, 0)
        return 0

    lax.fori_loop(0, EPW // SUP, super_body, 0)

    pltpu.make_async_copy(rows_a, acc_sh.at[dstc_a], ssem_a).wait()
    pltpu.make_async_copy(rows_b, acc_sh.at[dstc_b], ssem_b).wait()
    plsc.subcore_barrier()

    pltpu.sync_copy(acc_sh.at[pl.ds(s * RPT, RPT)],
                    num_hbm.at[c].at[pl.ds(s * RPT, RPT)])
    pltpu.sync_copy(den_v, den_hbm.at[wid])


def _sc_b(ei_flat, alpha, m, hw):
    f32 = jnp.float32
    return pl.kernel(
        _sc_b_body,
        out_type=[
            jax.ShapeDtypeStruct((NC, NP, D), f32),
            jax.ShapeDtypeStruct((NW, NP), f32),
        ],
        mesh=_mesh,
        compiler_params=pltpu.CompilerParams(needs_layout_passes=False),
        scratch_types=[
            pltpu.VMEM((NP,), f32),
            pltpu.VMEM((SUP,), jnp.int32),
            pltpu.VMEM((SUP,), jnp.int32),
            pltpu.VMEM((SUP,), f32),
            pltpu.VMEM((NP,), f32),
            pltpu.VMEM((CH,), f32),
            pltpu.VMEM((CH,), f32),
            pltpu.VMEM((CH,), jnp.int32),
            pltpu.VMEM((CH,), jnp.int32),
            pltpu.VMEM((CH, D), f32),
            pltpu.VMEM((CH, D), f32),
            pltpu.VMEM_SHARED((NP, D), f32),
            pltpu.SemaphoreType.DMA,
            pltpu.SemaphoreType.DMA,
            pltpu.SemaphoreType.DMA,
            pltpu.SemaphoreType.DMA,
        ],
    )(ei_flat, alpha, m, hw)


def kernel(x, edge_index, edge_attr, W1, b1, W2, b2, rel_emb, c1_W, c1_asrc,
           c1_adst, c1_We, c1_aedge, c1_b, c2_W, c2_asrc, c2_adst, c2_We,
           c2_aedge, c2_b, W3, b3, W4, b4):
    ei_flat = edge_index.reshape(2 * E)
    x_p = jnp.pad(x, ((0, NP - N), (0, 0)))

    hw, a_src, a_dst, t = _mlp_prep(x_p, W1, b1, W2, b2, c1_W, c1_asrc,
                                    c1_adst, rel_emb, c1_We, c1_aedge)

    for li, b in enumerate((c1_b, c2_b)):
        alpha, smax_p, tsum_p, cnt_p = _sc_a(
            ei_flat, edge_attr, a_src, a_dst, t)
        m, wloop = _combine(smax_p, tsum_p, cnt_p, a_src, a_dst)
        num_p, den_p = _sc_b(ei_flat, alpha, m, hw)
        if li == 0:
            hw, a_src, a_dst, t = _fin_prep(
                num_p, den_p, hw, wloop, b, c2_W, c2_asrc, c2_adst,
                rel_emb, c2_We, c2_aedge)
        else:
            out = _fin_pool(num_p, den_p, hw, wloop, b, W3, b3, W4, b4)
    return out
